# baseline (speedup 1.0000x reference)
import jax
import jax.numpy as jnp
from jax.experimental import pallas as pl
from jax.experimental.pallas import tpu as pltpu

_LANES = 128
_TILE_E = 32768
_GROUPS = _TILE_E // _LANES
_UNROLL_G = 8
_LBS = 8


def _fused_kernel(p_ref, q_ref, w2_ref, b2_ref, rows_hbm, lidx_ref, out_ref,
                  s_scr, t0, t1, t2s, t3, t4, t5, t6, t7, rows_smem, sems):
    j = pl.program_id(0)
    nsteps = pl.num_programs(0)
    half = jax.lax.rem(j, 2)

    def start_copy(s, h):
        src = rows_hbm.at[pl.ds(pl.multiple_of(s * _TILE_E, _TILE_E),
                                _TILE_E)]
        dst = rows_smem.at[pl.ds(pl.multiple_of(h * _TILE_E, _TILE_E),
                                 _TILE_E)]
        pltpu.make_async_copy(src, dst, sems.at[h]).start()

    @pl.when(j == 0)
    def _():
        start_copy(0, 0)

    @pl.when(j == 0)
    def _():
        n_nodes = p_ref.shape[0]
        nb = n_nodes // _LANES
        w2 = w2_ref[...]
        b2 = b2_ref[0, 0]
        ch = 512
        for c0 in range(0, n_nodes, ch):
            qc = q_ref[:, c0:c0 + ch]
            cb = c0 // _LANES

            def body(i, carry, qc=qc, cb=cb):
                for k in range(2):
                    a = i * 2 + k
                    pcol = p_ref[a].T
                    hid = jnp.maximum(qc + pcol, 0.0)
                    srow = jnp.sum(hid * w2, axis=0, keepdims=True) + b2
                    for p in range(ch // _LANES):
                        s_scr[pl.ds(a * nb + cb + p, 1)] = (
                            srow[:, None, p * _LANES:(p + 1) * _LANES])
                return carry

            jax.lax.fori_loop(0, n_nodes // 2, body, 0)

    cur = rows_smem.at[pl.ds(pl.multiple_of(half * _TILE_E, _TILE_E),
                             _TILE_E)]
    pltpu.make_async_copy(cur, cur, sems.at[half]).wait()

    @pl.when(j + 1 < nsteps)
    def _():
        start_copy(j + 1, 1 - half)

    base = half * _TILE_E
    sub_iota = jax.lax.broadcasted_iota(jnp.int32, (_LANES, _LANES), 0)
    tiles = (t0, t1, t2s, t3, t4, t5, t6, t7)

    def group_body(gi, carry):
        for k in range(_UNROLL_G):
            g = gi * _UNROLL_G + k
            tile = tiles[k]
            gbase = base + g * _LANES
            for c0 in range(0, _LANES, _LBS):
                vals = []
                for mi in range(c0, c0 + _LBS):
                    r = rows_smem[gbase + mi]
                    vals.append(s_scr[r])
                for i, mi in enumerate(range(c0, c0 + _LBS)):
                    tile[mi:mi + 1, :] = vals[i]
            t = tile[...]
            tt = t.T
            lrow = lidx_ref[g]
            vals2 = jnp.where(sub_iota == lrow, tt, 0.0)
            srow = jnp.sum(vals2, axis=0, keepdims=True)
            out_ref[pl.ds(g, 1)] = srow[:, None, :]
        return carry

    jax.lax.fori_loop(0, _GROUPS // _UNROLL_G, group_body, 0)


def kernel(h, src, dst, w1, b1, w2, b2):
    n, f = h.shape
    e = src.shape[0]

    w1a = w1[:, :f]
    w1b = w1[:, f:]
    pt = (w1a @ h.T) + b1.reshape(f, 1)
    qt = w1b @ h.T

    p3 = pt.T.reshape(n, 1, f)
    w2c = w2.reshape(f, 1).astype(jnp.float32)
    b2s = b2.reshape(1, 1).astype(jnp.float32)

    src_i = src.astype(jnp.int32)
    dst_i = dst.astype(jnp.int32)
    rows = src_i * (n // _LANES) + (dst_i >> 7)
    lidx = dst_i & (_LANES - 1)

    steps = max(1, -(-e // _TILE_E))
    epad = steps * _TILE_E
    rows_p = jnp.pad(rows, (0, epad - e))
    lidx_p = jnp.pad(lidx, (0, epad - e)).reshape(epad // _LANES, 1, _LANES)

    nrows = (n * n) // _LANES

    out = pl.pallas_call(
        _fused_kernel,
        out_shape=jax.ShapeDtypeStruct((epad // _LANES, 1, _LANES),
                                       jnp.float32),
        grid_spec=pltpu.PrefetchScalarGridSpec(
            num_scalar_prefetch=0,
            grid=(steps,),
            in_specs=[
                pl.BlockSpec((n, 1, f), lambda j: (0, 0, 0)),
                pl.BlockSpec((f, n), lambda j: (0, 0)),
                pl.BlockSpec((f, 1), lambda j: (0, 0)),
                pl.BlockSpec(memory_space=pltpu.SMEM),
                pl.BlockSpec(memory_space=pl.ANY),
                pl.BlockSpec((_GROUPS, 1, _LANES), lambda j: (j, 0, 0)),
            ],
            out_specs=pl.BlockSpec((_GROUPS, 1, _LANES),
                                   lambda j: (j, 0, 0)),
            scratch_shapes=(
                [pltpu.VMEM((nrows, 1, _LANES), jnp.float32)]
                + [pltpu.VMEM((_LANES, _LANES), jnp.float32)] * 8
                + [pltpu.SMEM((2 * _TILE_E,), jnp.int32),
                   pltpu.SemaphoreType.DMA((2,))]
            ),
        ),
        compiler_params=pltpu.CompilerParams(
            dimension_semantics=("arbitrary",),
            vmem_limit_bytes=int(26 << 20),
        ),
    )(p3, qt, w2c, b2s, rows_p, lidx_p)

    return out.reshape(epad)[:e]


# speedup vs baseline: 1.0868x; 1.0868x over previous
import jax
import jax.numpy as jnp
from jax.experimental import pallas as pl
from jax.experimental.pallas import tpu as pltpu

_LANES = 128
_TILE_E = 16384
_GROUPS = _TILE_E // _LANES
_UNROLL_G = 8
_LBS = 16


def _fused_kernel(p_ref, q_ref, w2_ref, b2_ref, rows_hbm, lidx_ref, out_ref,
                  s_scr, t0, t1, t2s, t3, t4, t5, t6, t7, rows_smem, sems):
    j = pl.program_id(0)
    nsteps = pl.num_programs(0)
    half = jax.lax.rem(j, 2)

    def start_copy(s, h):
        src = rows_hbm.at[pl.ds(pl.multiple_of(s * _TILE_E, _TILE_E),
                                _TILE_E)]
        dst = rows_smem.at[pl.ds(pl.multiple_of(h * _TILE_E, _TILE_E),
                                 _TILE_E)]
        pltpu.make_async_copy(src, dst, sems.at[h]).start()

    @pl.when(j == 0)
    def _():
        start_copy(0, 0)

    @pl.when(j == 0)
    def _():
        n_nodes = p_ref.shape[0]
        nb = n_nodes // _LANES
        q = q_ref[...]
        w2 = w2_ref[...]
        b2 = b2_ref[0, 0]

        def body(i, carry):
            for k in range(2):
                a = i * 2 + k
                pcol = p_ref[a].T
                hid = jnp.maximum(q + pcol, 0.0)
                srow = jnp.sum(hid * w2, axis=0, keepdims=True) + b2
                for p in range(nb):
                    s_scr[pl.ds(a * nb + p, 1)] = (
                        srow[:, None, p * _LANES:(p + 1) * _LANES])
            return carry

        jax.lax.fori_loop(0, n_nodes // 2, body, 0)

    cur = rows_smem.at[pl.ds(pl.multiple_of(half * _TILE_E, _TILE_E),
                             _TILE_E)]
    pltpu.make_async_copy(cur, cur, sems.at[half]).wait()

    @pl.when(j + 1 < nsteps)
    def _():
        start_copy(j + 1, 1 - half)

    base = half * _TILE_E
    sub_iota = jax.lax.broadcasted_iota(jnp.int32, (_LANES, _LANES), 0)
    tiles = (t0, t1, t2s, t3, t4, t5, t6, t7)

    def group_body(gi, carry):
        for k in range(_UNROLL_G):
            g = gi * _UNROLL_G + k
            tile = tiles[k]
            gbase = base + g * _LANES
            for c0 in range(0, _LANES, _LBS):
                vals = []
                for mi in range(c0, c0 + _LBS):
                    r = rows_smem[gbase + mi]
                    vals.append(s_scr[r])
                for i, mi in enumerate(range(c0, c0 + _LBS)):
                    tile[mi:mi + 1, :] = vals[i]
            t = tile[...]
            tt = t.T
            lrow = lidx_ref[g]
            vals2 = jnp.where(sub_iota == lrow, tt, 0.0)
            srow = jnp.sum(vals2, axis=0, keepdims=True)
            out_ref[pl.ds(g, 1)] = srow[:, None, :]
        return carry

    jax.lax.fori_loop(0, _GROUPS // _UNROLL_G, group_body, 0)


def kernel(h, src, dst, w1, b1, w2, b2):
    n, f = h.shape
    e = src.shape[0]

    w1a = w1[:, :f]
    w1b = w1[:, f:]
    pt = (w1a @ h.T) + b1.reshape(f, 1)
    qt = w1b @ h.T

    p3 = pt.T.reshape(n, 1, f)
    w2c = w2.reshape(f, 1).astype(jnp.float32)
    b2s = b2.reshape(1, 1).astype(jnp.float32)

    src_i = src.astype(jnp.int32)
    dst_i = dst.astype(jnp.int32)
    rows = src_i * (n // _LANES) + (dst_i >> 7)
    lidx = dst_i & (_LANES - 1)

    steps = max(1, -(-e // _TILE_E))
    epad = steps * _TILE_E
    rows_p = jnp.pad(rows, (0, epad - e))
    lidx_p = jnp.pad(lidx, (0, epad - e)).reshape(epad // _LANES, 1, _LANES)

    nrows = (n * n) // _LANES

    out = pl.pallas_call(
        _fused_kernel,
        out_shape=jax.ShapeDtypeStruct((epad // _LANES, 1, _LANES),
                                       jnp.float32),
        grid_spec=pltpu.PrefetchScalarGridSpec(
            num_scalar_prefetch=0,
            grid=(steps,),
            in_specs=[
                pl.BlockSpec((n, 1, f), lambda j: (0, 0, 0)),
                pl.BlockSpec((f, n), lambda j: (0, 0)),
                pl.BlockSpec((f, 1), lambda j: (0, 0)),
                pl.BlockSpec(memory_space=pltpu.SMEM),
                pl.BlockSpec(memory_space=pl.ANY),
                pl.BlockSpec((_GROUPS, 1, _LANES), lambda j: (j, 0, 0)),
            ],
            out_specs=pl.BlockSpec((_GROUPS, 1, _LANES),
                                   lambda j: (j, 0, 0)),
            scratch_shapes=(
                [pltpu.VMEM((nrows, 1, _LANES), jnp.float32)]
                + [pltpu.VMEM((_LANES, _LANES), jnp.float32)] * 8
                + [pltpu.SMEM((2 * _TILE_E,), jnp.int32),
                   pltpu.SemaphoreType.DMA((2,))]
            ),
        ),
        compiler_params=pltpu.CompilerParams(
            dimension_semantics=("arbitrary",),
            vmem_limit_bytes=int(26 << 20),
        ),
    )(p3, qt, w2c, b2s, rows_p, lidx_p)

    return out.reshape(epad)[:e]


# speedup vs baseline: 1.1208x; 1.0313x over previous
import jax
import jax.numpy as jnp
from jax.experimental import pallas as pl
from jax.experimental.pallas import tpu as pltpu

_LANES = 128
_TILE_E = 16384
_GROUPS = _TILE_E // _LANES
_UNROLL_G = 16
_LBS = 16


def _fused_kernel(p_ref, q_ref, w2_ref, b2_ref, rows_hbm, lidx_ref, out_ref,
                  s_scr, t0, t1, t2s, t3, t4, t5, t6, t7,
                  t8, t9, t10, t11, t12, t13, t14, t15, rows_smem, sems):
    j = pl.program_id(0)
    nsteps = pl.num_programs(0)
    half = jax.lax.rem(j, 2)

    def start_copy(s, h):
        src = rows_hbm.at[pl.ds(pl.multiple_of(s * _TILE_E, _TILE_E),
                                _TILE_E)]
        dst = rows_smem.at[pl.ds(pl.multiple_of(h * _TILE_E, _TILE_E),
                                 _TILE_E)]
        pltpu.make_async_copy(src, dst, sems.at[h]).start()

    @pl.when(j == 0)
    def _():
        start_copy(0, 0)

    @pl.when(j == 0)
    def _():
        n_nodes = p_ref.shape[0]
        nb = n_nodes // _LANES
        q = q_ref[...]
        w2 = w2_ref[...]
        b2 = b2_ref[0, 0]

        def body(i, carry):
            for k in range(2):
                a = i * 2 + k
                pcol = p_ref[a].T
                hid = jnp.maximum(q + pcol, 0.0)
                srow = jnp.sum(hid * w2, axis=0, keepdims=True) + b2
                for p in range(nb):
                    s_scr[pl.ds(a * nb + p, 1)] = (
                        srow[:, None, p * _LANES:(p + 1) * _LANES])
            return carry

        jax.lax.fori_loop(0, n_nodes // 2, body, 0)

    cur = rows_smem.at[pl.ds(pl.multiple_of(half * _TILE_E, _TILE_E),
                             _TILE_E)]
    pltpu.make_async_copy(cur, cur, sems.at[half]).wait()

    @pl.when(j + 1 < nsteps)
    def _():
        start_copy(j + 1, 1 - half)

    base = half * _TILE_E
    sub_iota = jax.lax.broadcasted_iota(jnp.int32, (_LANES, _LANES), 0)
    tiles = (t0, t1, t2s, t3, t4, t5, t6, t7,
             t8, t9, t10, t11, t12, t13, t14, t15)

    def group_body(gi, carry):
        for k in range(_UNROLL_G):
            g = gi * _UNROLL_G + k
            tile = tiles[k]
            gbase = base + g * _LANES
            for c0 in range(0, _LANES, _LBS):
                vals = []
                for mi in range(c0, c0 + _LBS):
                    r = rows_smem[gbase + mi]
                    vals.append(s_scr[r])
                for i, mi in enumerate(range(c0, c0 + _LBS)):
                    tile[mi:mi + 1, :] = vals[i]
            t = tile[...]
            tt = t.T
            lrow = lidx_ref[g]
            vals2 = jnp.where(sub_iota == lrow, tt, 0.0)
            srow = jnp.sum(vals2, axis=0, keepdims=True)
            out_ref[pl.ds(g, 1)] = srow[:, None, :]
        return carry

    jax.lax.fori_loop(0, _GROUPS // _UNROLL_G, group_body, 0)


def kernel(h, src, dst, w1, b1, w2, b2):
    n, f = h.shape
    e = src.shape[0]

    w1a = w1[:, :f]
    w1b = w1[:, f:]
    pt = (w1a @ h.T) + b1.reshape(f, 1)
    qt = w1b @ h.T

    p3 = pt.T.reshape(n, 1, f)
    w2c = w2.reshape(f, 1).astype(jnp.float32)
    b2s = b2.reshape(1, 1).astype(jnp.float32)

    src_i = src.astype(jnp.int32)
    dst_i = dst.astype(jnp.int32)
    rows = src_i * (n // _LANES) + (dst_i >> 7)
    lidx = dst_i & (_LANES - 1)

    steps = max(1, -(-e // _TILE_E))
    epad = steps * _TILE_E
    rows_p = jnp.pad(rows, (0, epad - e))
    lidx_p = jnp.pad(lidx, (0, epad - e)).reshape(epad // _LANES, 1, _LANES)

    nrows = (n * n) // _LANES

    out = pl.pallas_call(
        _fused_kernel,
        out_shape=jax.ShapeDtypeStruct((epad // _LANES, 1, _LANES),
                                       jnp.float32),
        grid_spec=pltpu.PrefetchScalarGridSpec(
            num_scalar_prefetch=0,
            grid=(steps,),
            in_specs=[
                pl.BlockSpec((n, 1, f), lambda j: (0, 0, 0)),
                pl.BlockSpec((f, n), lambda j: (0, 0)),
                pl.BlockSpec((f, 1), lambda j: (0, 0)),
                pl.BlockSpec(memory_space=pltpu.SMEM),
                pl.BlockSpec(memory_space=pl.ANY),
                pl.BlockSpec((_GROUPS, 1, _LANES), lambda j: (j, 0, 0)),
            ],
            out_specs=pl.BlockSpec((_GROUPS, 1, _LANES),
                                   lambda j: (j, 0, 0)),
            scratch_shapes=(
                [pltpu.VMEM((nrows, 1, _LANES), jnp.float32)]
                + [pltpu.VMEM((_LANES, _LANES), jnp.float32)] * 16
                + [pltpu.SMEM((2 * _TILE_E,), jnp.int32),
                   pltpu.SemaphoreType.DMA((2,))]
            ),
        ),
        compiler_params=pltpu.CompilerParams(
            dimension_semantics=("arbitrary",),
            vmem_limit_bytes=int(26 << 20),
        ),
    )(p3, qt, w2c, b2s, rows_p, lidx_p)

    return out.reshape(epad)[:e]


# speedup vs baseline: 1.1361x; 1.0137x over previous
import jax
import jax.numpy as jnp
from jax.experimental import pallas as pl
from jax.experimental.pallas import tpu as pltpu

_LANES = 128
_TILE_E = 16384
_GROUPS = _TILE_E // _LANES
_UNROLL_G = 32
_LBS = 16


def _fused_kernel(p_ref, q_ref, w2_ref, b2_ref, rows_hbm, lidx_ref, out_ref,
                  s_scr, *rest):
    tiles = rest[:_UNROLL_G]
    rows_smem = rest[_UNROLL_G]
    sems = rest[_UNROLL_G + 1]
    j = pl.program_id(0)
    nsteps = pl.num_programs(0)
    half = jax.lax.rem(j, 2)

    def start_copy(s, h):
        src = rows_hbm.at[pl.ds(pl.multiple_of(s * _TILE_E, _TILE_E),
                                _TILE_E)]
        dst = rows_smem.at[pl.ds(pl.multiple_of(h * _TILE_E, _TILE_E),
                                 _TILE_E)]
        pltpu.make_async_copy(src, dst, sems.at[h]).start()

    @pl.when(j == 0)
    def _():
        start_copy(0, 0)

    @pl.when(j == 0)
    def _():
        n_nodes = p_ref.shape[0]
        nb = n_nodes // _LANES
        q = q_ref[...]
        w2 = w2_ref[...]
        b2 = b2_ref[0, 0]

        def body(i, carry):
            for k in range(2):
                a = i * 2 + k
                pcol = p_ref[a].T
                hid = jnp.maximum(q + pcol, 0.0)
                srow = jnp.sum(hid * w2, axis=0, keepdims=True) + b2
                for p in range(nb):
                    s_scr[pl.ds(a * nb + p, 1)] = (
                        srow[:, None, p * _LANES:(p + 1) * _LANES])
            return carry

        jax.lax.fori_loop(0, n_nodes // 2, body, 0)

    cur = rows_smem.at[pl.ds(pl.multiple_of(half * _TILE_E, _TILE_E),
                             _TILE_E)]
    pltpu.make_async_copy(cur, cur, sems.at[half]).wait()

    @pl.when(j + 1 < nsteps)
    def _():
        start_copy(j + 1, 1 - half)

    base = half * _TILE_E
    sub_iota = jax.lax.broadcasted_iota(jnp.int32, (_LANES, _LANES), 0)

    def group_body(gi, carry):
        for k in range(_UNROLL_G):
            g = gi * _UNROLL_G + k
            tile = tiles[k]
            gbase = base + g * _LANES
            for c0 in range(0, _LANES, _LBS):
                vals = []
                for mi in range(c0, c0 + _LBS):
                    r = rows_smem[gbase + mi]
                    vals.append(s_scr[r])
                for i, mi in enumerate(range(c0, c0 + _LBS)):
                    tile[mi:mi + 1, :] = vals[i]
            t = tile[...]
            tt = t.T
            lrow = lidx_ref[g]
            vals2 = jnp.where(sub_iota == lrow, tt, 0.0)
            srow = jnp.sum(vals2, axis=0, keepdims=True)
            out_ref[pl.ds(g, 1)] = srow[:, None, :]
        return carry

    jax.lax.fori_loop(0, _GROUPS // _UNROLL_G, group_body, 0)


def kernel(h, src, dst, w1, b1, w2, b2):
    n, f = h.shape
    e = src.shape[0]

    w1a = w1[:, :f]
    w1b = w1[:, f:]
    pt = (w1a @ h.T) + b1.reshape(f, 1)
    qt = w1b @ h.T

    p3 = pt.T.reshape(n, 1, f)
    w2c = w2.reshape(f, 1).astype(jnp.float32)
    b2s = b2.reshape(1, 1).astype(jnp.float32)

    src_i = src.astype(jnp.int32)
    dst_i = dst.astype(jnp.int32)
    rows = src_i * (n // _LANES) + (dst_i >> 7)
    lidx = dst_i & (_LANES - 1)

    steps = max(1, -(-e // _TILE_E))
    epad = steps * _TILE_E
    rows_p = jnp.pad(rows, (0, epad - e))
    lidx_p = jnp.pad(lidx, (0, epad - e)).reshape(epad // _LANES, 1, _LANES)

    nrows = (n * n) // _LANES

    out = pl.pallas_call(
        _fused_kernel,
        out_shape=jax.ShapeDtypeStruct((epad // _LANES, 1, _LANES),
                                       jnp.float32),
        grid_spec=pltpu.PrefetchScalarGridSpec(
            num_scalar_prefetch=0,
            grid=(steps,),
            in_specs=[
                pl.BlockSpec((n, 1, f), lambda j: (0, 0, 0)),
                pl.BlockSpec((f, n), lambda j: (0, 0)),
                pl.BlockSpec((f, 1), lambda j: (0, 0)),
                pl.BlockSpec(memory_space=pltpu.SMEM),
                pl.BlockSpec(memory_space=pl.ANY),
                pl.BlockSpec((_GROUPS, 1, _LANES), lambda j: (j, 0, 0)),
            ],
            out_specs=pl.BlockSpec((_GROUPS, 1, _LANES),
                                   lambda j: (j, 0, 0)),
            scratch_shapes=(
                [pltpu.VMEM((nrows, 1, _LANES), jnp.float32)]
                + [pltpu.VMEM((_LANES, _LANES), jnp.float32)] * _UNROLL_G
                + [pltpu.SMEM((2 * _TILE_E,), jnp.int32),
                   pltpu.SemaphoreType.DMA((2,))]
            ),
        ),
        compiler_params=pltpu.CompilerParams(
            dimension_semantics=("arbitrary",),
            vmem_limit_bytes=int(26 << 20),
        ),
    )(p3, qt, w2c, b2s, rows_p, lidx_p)

    return out.reshape(epad)[:e]


# speedup vs baseline: 1.1470x; 1.0095x over previous
import jax
import jax.numpy as jnp
from jax.experimental import pallas as pl
from jax.experimental.pallas import tpu as pltpu

_LANES = 128
_TILE_E = 16384
_GROUPS = _TILE_E // _LANES
_UNROLL_G = 64
_LBS = 16


def _fused_kernel(p_ref, q_ref, w2_ref, b2_ref, rows_hbm, lidx_ref, out_ref,
                  s_scr, *rest):
    tiles = rest[:_UNROLL_G]
    rows_smem = rest[_UNROLL_G]
    sems = rest[_UNROLL_G + 1]
    j = pl.program_id(0)
    nsteps = pl.num_programs(0)
    half = jax.lax.rem(j, 2)

    def start_copy(s, h):
        src = rows_hbm.at[pl.ds(pl.multiple_of(s * _TILE_E, _TILE_E),
                                _TILE_E)]
        dst = rows_smem.at[pl.ds(pl.multiple_of(h * _TILE_E, _TILE_E),
                                 _TILE_E)]
        pltpu.make_async_copy(src, dst, sems.at[h]).start()

    @pl.when(j == 0)
    def _():
        start_copy(0, 0)

    @pl.when(j == 0)
    def _():
        n_nodes = p_ref.shape[0]
        nb = n_nodes // _LANES
        q = q_ref[...]
        w2 = w2_ref[...]
        b2 = b2_ref[0, 0]

        def body(i, carry):
            for k in range(2):
                a = i * 2 + k
                pcol = p_ref[a].T
                hid = jnp.maximum(q + pcol, 0.0)
                srow = jnp.sum(hid * w2, axis=0, keepdims=True) + b2
                for p in range(nb):
                    s_scr[pl.ds(a * nb + p, 1)] = (
                        srow[:, None, p * _LANES:(p + 1) * _LANES])
            return carry

        jax.lax.fori_loop(0, n_nodes // 2, body, 0)

    cur = rows_smem.at[pl.ds(pl.multiple_of(half * _TILE_E, _TILE_E),
                             _TILE_E)]
    pltpu.make_async_copy(cur, cur, sems.at[half]).wait()

    @pl.when(j + 1 < nsteps)
    def _():
        start_copy(j + 1, 1 - half)

    base = half * _TILE_E
    sub_iota = jax.lax.broadcasted_iota(jnp.int32, (_LANES, _LANES), 0)

    def group_body(gi, carry):
        for k in range(_UNROLL_G):
            g = gi * _UNROLL_G + k
            tile = tiles[k]
            gbase = base + g * _LANES
            for c0 in range(0, _LANES, _LBS):
                vals = []
                for mi in range(c0, c0 + _LBS):
                    r = rows_smem[gbase + mi]
                    vals.append(s_scr[r])
                for i, mi in enumerate(range(c0, c0 + _LBS)):
                    tile[mi:mi + 1, :] = vals[i]
            t = tile[...]
            tt = t.T
            lrow = lidx_ref[g]
            vals2 = jnp.where(sub_iota == lrow, tt, 0.0)
            srow = jnp.sum(vals2, axis=0, keepdims=True)
            out_ref[pl.ds(g, 1)] = srow[:, None, :]
        return carry

    jax.lax.fori_loop(0, _GROUPS // _UNROLL_G, group_body, 0)


def kernel(h, src, dst, w1, b1, w2, b2):
    n, f = h.shape
    e = src.shape[0]

    w1a = w1[:, :f]
    w1b = w1[:, f:]
    pt = (w1a @ h.T) + b1.reshape(f, 1)
    qt = w1b @ h.T

    p3 = pt.T.reshape(n, 1, f)
    w2c = w2.reshape(f, 1).astype(jnp.float32)
    b2s = b2.reshape(1, 1).astype(jnp.float32)

    src_i = src.astype(jnp.int32)
    dst_i = dst.astype(jnp.int32)
    rows = src_i * (n // _LANES) + (dst_i >> 7)
    lidx = dst_i & (_LANES - 1)

    steps = max(1, -(-e // _TILE_E))
    epad = steps * _TILE_E
    rows_p = jnp.pad(rows, (0, epad - e))
    lidx_p = jnp.pad(lidx, (0, epad - e)).reshape(epad // _LANES, 1, _LANES)

    nrows = (n * n) // _LANES

    out = pl.pallas_call(
        _fused_kernel,
        out_shape=jax.ShapeDtypeStruct((epad // _LANES, 1, _LANES),
                                       jnp.float32),
        grid_spec=pltpu.PrefetchScalarGridSpec(
            num_scalar_prefetch=0,
            grid=(steps,),
            in_specs=[
                pl.BlockSpec((n, 1, f), lambda j: (0, 0, 0)),
                pl.BlockSpec((f, n), lambda j: (0, 0)),
                pl.BlockSpec((f, 1), lambda j: (0, 0)),
                pl.BlockSpec(memory_space=pltpu.SMEM),
                pl.BlockSpec(memory_space=pl.ANY),
                pl.BlockSpec((_GROUPS, 1, _LANES), lambda j: (j, 0, 0)),
            ],
            out_specs=pl.BlockSpec((_GROUPS, 1, _LANES),
                                   lambda j: (j, 0, 0)),
            scratch_shapes=(
                [pltpu.VMEM((nrows, 1, _LANES), jnp.float32)]
                + [pltpu.VMEM((_LANES, _LANES), jnp.float32)] * _UNROLL_G
                + [pltpu.SMEM((2 * _TILE_E,), jnp.int32),
                   pltpu.SemaphoreType.DMA((2,))]
            ),
        ),
        compiler_params=pltpu.CompilerParams(
            dimension_semantics=("arbitrary",),
            vmem_limit_bytes=int(26 << 20),
        ),
    )(p3, qt, w2c, b2s, rows_p, lidx_p)

    return out.reshape(epad)[:e]
